# speedup vs baseline: 4.4332x; 4.4332x over previous
import jax
import jax.numpy as jnp
from jax.experimental import pallas as pl


def _tc_body(z_ref, x_ref, sc_ref, sh_ref, o_ref):
    z = z_ref[0, 0, :]
    b = z.shape[0]
    t = sc_ref.shape[0]
    oh = (z[:, None] == jax.lax.broadcasted_iota(jnp.int32, (b, t), 1)).astype(
        jnp.float32
    )
    scale = jnp.dot(oh, sc_ref[...], preferred_element_type=jnp.float32)
    shift = jnp.dot(oh, sh_ref[...], preferred_element_type=jnp.float32)
    o_ref[...] = x_ref[...] * scale + shift


def kernel(inputs, z, scale_w, shift_w):
    n, d = inputs.shape
    t = scale_w.shape[0]
    tp = 128
    scale_p = jnp.zeros((tp, d), jnp.float32).at[:t].set(scale_w)
    shift_p = jnp.zeros((tp, d), jnp.float32).at[:t].set(shift_w)
    B = 1000
    nblk = n // B
    z3 = z.astype(jnp.int32).reshape(nblk, 1, B)
    return pl.pallas_call(
        _tc_body,
        grid=(nblk,),
        in_specs=[
            pl.BlockSpec((1, 1, B), lambda i: (i, 0, 0)),
            pl.BlockSpec((B, d), lambda i: (i, 0)),
            pl.BlockSpec((tp, d), lambda i: (0, 0)),
            pl.BlockSpec((tp, d), lambda i: (0, 0)),
        ],
        out_specs=pl.BlockSpec((B, d), lambda i: (i, 0)),
        out_shape=jax.ShapeDtypeStruct((n, d), jnp.float32),
    )(z3, inputs, scale_p, shift_p)
